# Initial kernel scaffold; baseline (speedup 1.0000x reference)
import jax
import jax.numpy as jnp
from jax.experimental import pallas as pl


def _probe_body(tab_ref, idx_ref, out_ref):
    tab = tab_ref[...]
    idx = idx_ref[...]
    out_ref[...] = jnp.take_along_axis(tab, idx, axis=1)


def kernel(x, params, k):
    B, C, N = x.shape
    tab = jnp.zeros((64, 1024), jnp.float32) + x[0, 0, 0]
    idx = jnp.zeros((64, 256), jnp.int32)
    out = pl.pallas_call(
        _probe_body,
        out_shape=jax.ShapeDtypeStruct((64, 256), jnp.float32),
    )(tab, idx)
    return jnp.zeros((8, 40), jnp.float32) + out[0, 0]


# speedup vs baseline: 1.0010x; 1.0010x over previous
import jax
import jax.numpy as jnp
from jax.experimental import pallas as pl


def _lrelu(x):
    return jnp.where(x >= 0, x, 0.2 * x)


def _bn(x, g, b, ch_axis):
    axes = tuple(i for i in range(x.ndim) if i != ch_axis)
    mean = jnp.mean(x, axis=axes, keepdims=True)
    var = jnp.var(x, axis=axes, keepdims=True)
    shp = [1] * x.ndim
    shp[ch_axis] = -1
    return (x - mean) / jnp.sqrt(var + 1e-5) * g.reshape(shp) + b.reshape(shp)


def _knn(x, k):
    inner = -2.0 * jnp.einsum('bcn,bcm->bnm', x, x)
    xx = jnp.sum(x * x, axis=1)
    pd = -xx[:, :, None] - inner - xx[:, None, :]
    _, idx = jax.lax.top_k(pd, k)
    return idx


def _graph_feature(x, k, firstlayer):
    B, C, N = x.shape
    idx = _knn(x, k)
    x_t = jnp.transpose(x, (0, 2, 1))
    flat = x_t.reshape(B * N, C)
    idxf = (idx + (jnp.arange(B) * N)[:, None, None]).reshape(-1)
    feature = flat[idxf].reshape(B, N, k, C)
    xr = jnp.broadcast_to(x_t[:, :, None, :], (B, N, k, C))
    d = feature - xr
    ded3 = jnp.sum(d * d, axis=-1, keepdims=True)
    x_glo = jnp.concatenate([jnp.sum(x, axis=1), jnp.mean(x, axis=1)], axis=-1)
    if firstlayer:
        f = jnp.concatenate([ded3, d, xr, feature], axis=3)
    else:
        f = jnp.concatenate([d, xr], axis=3)
    f = jnp.transpose(f, (0, 3, 1, 2))
    return x[:, :, :, None], f, x_glo


def _lab(xq, xk, p):
    q = _lrelu(_bn(jnp.einsum('oc,bcnk->bonk', p['Wk'], xq), p['gk'], p['bk'], 1))
    v = _lrelu(_bn(jnp.einsum('oc,bcnk->bonk', p['Wv'], xk), p['gv'], p['bv'], 1))
    s = _lrelu(_bn(jnp.einsum('oc,bcnk->bonk', p['Ws'], q + v), p['gs'], p['bs'], 1))
    s = jax.nn.softmax(s[:, 0], axis=2)
    return jnp.sum(v * s[:, None, :, :], axis=-1)


def _gab(xq, xk, p):
    values = _lrelu(_bn(xk @ p['W1'].T, p['g1'], p['b1'], 2))
    feats = values + xq[:, None, :]
    s = _lrelu(_bn(feats @ p['W2'].T, p['g2'], p['b2'], 2))
    s = _lrelu(_bn(s @ p['W3'].T, p['g3'], p['b3'], 2))
    s = _lrelu(_bn(s @ p['W4'].T, p['g4'], p['b4'], 2))
    s = jax.nn.softmax(s[..., 0], axis=1)
    return jnp.sum(values * s[:, :, None], axis=1)


def _ident_body(x_ref, o_ref):
    o_ref[...] = x_ref[...]


def kernel(x, params, k):
    x = pl.pallas_call(
        _ident_body, out_shape=jax.ShapeDtypeStruct(x.shape, x.dtype))(x)
    ver1, fea1, x_glo = _graph_feature(x, 20, True)
    x1 = _lab(ver1, fea1, params['lab1'])
    ver2, fea2, _ = _graph_feature(x1, 20, False)
    x2 = _lab(ver2, fea2, params['lab2'])
    ver3, fea3, _ = _graph_feature(x2, 20, False)
    x3 = _lab(ver3, fea3, params['lab3'])
    ver4, fea4, _ = _graph_feature(x3, 20, False)
    x4 = _lab(ver4, fea4, params['lab4'])
    xc = jnp.concatenate([x1, x2, x3, x4], axis=1)
    xk = jnp.transpose(xc, (0, 2, 1))
    g = _gab(x_glo, xk, params['gab'])
    h = _lrelu(_bn(g @ params['WL1'].T, params['g6'], params['b6'], 1))
    h = _lrelu(_bn(h @ params['WL2'].T + params['bL2'], params['g7'], params['b7'], 1))
    return h @ params['WL3'].T + params['bL3']


# speedup vs baseline: 1.2600x; 1.2587x over previous
import functools

import jax
import jax.numpy as jnp
from jax import lax
from jax.experimental import pallas as pl
from jax.experimental.pallas import tpu as pltpu
from jax.experimental.pallas import tpu_sc as plsc

_NW = 32


def _sc_gather(table, idx):
    V, D = table.shape
    E = idx.shape[0]
    e_per_w = E // _NW
    ch = max(8, min(128, (1 << 15) // D))
    while e_per_w % ch:
        ch //= 2
    steps = e_per_w // ch
    mesh = plsc.VectorSubcoreMesh(core_axis_name="c", subcore_axis_name="s")

    @functools.partial(
        pl.kernel,
        out_type=jax.ShapeDtypeStruct((E, D), jnp.float32),
        mesh=mesh,
        scratch_types=[
            pltpu.VMEM((ch,), jnp.int32),
            pltpu.VMEM((ch, D), jnp.float32),
            pltpu.SemaphoreType.DMA,
        ],
        compiler_params=pltpu.CompilerParams(use_tc_tiling_on_sc=False),
    )
    def k(table_hbm, idx_hbm, out_hbm, idx_v, rows_v, sem):
        wid = lax.axis_index("s") * 2 + lax.axis_index("c")
        base = wid * e_per_w

        def body(i, carry):
            off = base + i * ch
            pltpu.sync_copy(idx_hbm.at[pl.ds(off, ch)], idx_v)
            pltpu.async_copy(table_hbm.at[idx_v], rows_v, sem).wait()
            pltpu.sync_copy(rows_v, out_hbm.at[pl.ds(off, ch)])
            return carry

        lax.fori_loop(0, steps, body, 0)

    return k(table, idx)


def _lrelu(x):
    return jnp.where(x >= 0, x, 0.2 * x)


def _bn(x, g, b, ch_axis):
    axes = tuple(i for i in range(x.ndim) if i != ch_axis)
    mean = jnp.mean(x, axis=axes, keepdims=True)
    var = jnp.var(x, axis=axes, keepdims=True)
    shp = [1] * x.ndim
    shp[ch_axis] = -1
    return (x - mean) / jnp.sqrt(var + 1e-5) * g.reshape(shp) + b.reshape(shp)


def _knn(x, k):
    inner = -2.0 * jnp.einsum('bcn,bcm->bnm', x, x)
    xx = jnp.sum(x * x, axis=1)
    pd = -xx[:, :, None] - inner - xx[:, None, :]
    _, idx = jax.lax.top_k(pd, k)
    return idx


def _graph_feature(x, k, firstlayer):
    B, C, N = x.shape
    idx = _knn(x, k)
    x_t = jnp.transpose(x, (0, 2, 1))
    flat = x_t.reshape(B * N, C)
    idxf = (idx + (jnp.arange(B) * N)[:, None, None]).reshape(-1).astype(jnp.int32)
    Dp = max(16, -(-C // 16) * 16)
    flat_p = jnp.pad(flat, ((0, 0), (0, Dp - C)))
    feature = _sc_gather(flat_p, idxf)[:, :C].reshape(B, N, k, C)
    xr = jnp.broadcast_to(x_t[:, :, None, :], (B, N, k, C))
    d = feature - xr
    ded3 = jnp.sum(d * d, axis=-1, keepdims=True)
    x_glo = jnp.concatenate([jnp.sum(x, axis=1), jnp.mean(x, axis=1)], axis=-1)
    if firstlayer:
        f = jnp.concatenate([ded3, d, xr, feature], axis=3)
    else:
        f = jnp.concatenate([d, xr], axis=3)
    f = jnp.transpose(f, (0, 3, 1, 2))
    return x[:, :, :, None], f, x_glo


def _lab(xq, xk, p):
    q = _lrelu(_bn(jnp.einsum('oc,bcnk->bonk', p['Wk'], xq), p['gk'], p['bk'], 1))
    v = _lrelu(_bn(jnp.einsum('oc,bcnk->bonk', p['Wv'], xk), p['gv'], p['bv'], 1))
    s = _lrelu(_bn(jnp.einsum('oc,bcnk->bonk', p['Ws'], q + v), p['gs'], p['bs'], 1))
    s = jax.nn.softmax(s[:, 0], axis=2)
    return jnp.sum(v * s[:, None, :, :], axis=-1)


def _gab(xq, xk, p):
    values = _lrelu(_bn(xk @ p['W1'].T, p['g1'], p['b1'], 2))
    feats = values + xq[:, None, :]
    s = _lrelu(_bn(feats @ p['W2'].T, p['g2'], p['b2'], 2))
    s = _lrelu(_bn(s @ p['W3'].T, p['g3'], p['b3'], 2))
    s = _lrelu(_bn(s @ p['W4'].T, p['g4'], p['b4'], 2))
    s = jax.nn.softmax(s[..., 0], axis=1)
    return jnp.sum(values * s[:, :, None], axis=1)


def kernel(x, params, k):
    ver1, fea1, x_glo = _graph_feature(x, 20, True)
    x1 = _lab(ver1, fea1, params['lab1'])
    ver2, fea2, _ = _graph_feature(x1, 20, False)
    x2 = _lab(ver2, fea2, params['lab2'])
    ver3, fea3, _ = _graph_feature(x2, 20, False)
    x3 = _lab(ver3, fea3, params['lab3'])
    ver4, fea4, _ = _graph_feature(x3, 20, False)
    x4 = _lab(ver4, fea4, params['lab4'])
    xc = jnp.concatenate([x1, x2, x3, x4], axis=1)
    xk = jnp.transpose(xc, (0, 2, 1))
    g = _gab(x_glo, xk, params['gab'])
    h = _lrelu(_bn(g @ params['WL1'].T, params['g6'], params['b6'], 1))
    h = _lrelu(_bn(h @ params['WL2'].T + params['bL2'], params['g7'], params['b7'], 1))
    return h @ params['WL3'].T + params['bL3']


# speedup vs baseline: 3.0168x; 2.3943x over previous
import functools

import jax
import jax.numpy as jnp
from jax import lax
from jax.experimental import pallas as pl
from jax.experimental.pallas import tpu as pltpu
from jax.experimental.pallas import tpu_sc as plsc

_NW = 32


def _sc_gather(table, idx):
    V, D = table.shape
    E = idx.shape[0]
    e_per_w = E // _NW
    ch = max(8, min(128, (1 << 15) // D))
    while e_per_w % ch:
        ch //= 2
    steps = e_per_w // ch
    mesh = plsc.VectorSubcoreMesh(core_axis_name="c", subcore_axis_name="s")

    @functools.partial(
        pl.kernel,
        out_type=jax.ShapeDtypeStruct((E, D), jnp.float32),
        mesh=mesh,
        scratch_types=[
            pltpu.VMEM((ch,), jnp.int32),
            pltpu.VMEM((ch, D), jnp.float32),
            pltpu.SemaphoreType.DMA,
        ],
        compiler_params=pltpu.CompilerParams(use_tc_tiling_on_sc=False),
    )
    def k(table_hbm, idx_hbm, out_hbm, idx_v, rows_v, sem):
        wid = lax.axis_index("s") * 2 + lax.axis_index("c")
        base = wid * e_per_w

        def body(i, carry):
            off = base + i * ch
            pltpu.sync_copy(idx_hbm.at[pl.ds(off, ch)], idx_v)
            pltpu.async_copy(table_hbm.at[idx_v], rows_v, sem).wait()
            pltpu.sync_copy(rows_v, out_hbm.at[pl.ds(off, ch)])
            return carry

        lax.fori_loop(0, steps, body, 0)

    return k(table, idx)


def _lrelu(x):
    return jnp.where(x >= 0, x, 0.2 * x)


def _bn(x, g, b, ch_axis):
    axes = tuple(i for i in range(x.ndim) if i != ch_axis)
    mean = jnp.mean(x, axis=axes, keepdims=True)
    var = jnp.var(x, axis=axes, keepdims=True)
    shp = [1] * x.ndim
    shp[ch_axis] = -1
    return (x - mean) / jnp.sqrt(var + 1e-5) * g.reshape(shp) + b.reshape(shp)


def _topk_body(pd_ref, idx_ref):
    cur = pd_ref[...]
    iota = lax.broadcasted_iota(jnp.int32, cur.shape, 1)
    big = jnp.int32(1 << 30)
    outs = []
    for _ in range(20):
        m = jnp.max(cur, axis=1, keepdims=True)
        am = jnp.min(jnp.where(cur == m, iota, big), axis=1, keepdims=True)
        outs.append(am)
        cur = jnp.where(iota == am, -jnp.inf, cur)
    idx_ref[...] = jnp.concatenate(outs, axis=1)


def _topk20(pd):
    B, N, M = pd.shape
    PT = 256
    flat = pd.reshape(B * N, M)
    idx = pl.pallas_call(
        _topk_body,
        grid=(B * N // PT,),
        in_specs=[pl.BlockSpec((PT, M), lambda i: (i, 0))],
        out_specs=pl.BlockSpec((PT, 20), lambda i: (i, 0)),
        out_shape=jax.ShapeDtypeStruct((B * N, 20), jnp.int32),
    )(flat)
    return idx.reshape(B, N, 20)


def _knn(x, k):
    inner = -2.0 * jnp.einsum('bcn,bcm->bnm', x, x)
    xx = jnp.sum(x * x, axis=1)
    pd = -xx[:, :, None] - inner - xx[:, None, :]
    return _topk20(pd)


def _graph_feature(x, k, firstlayer):
    B, C, N = x.shape
    idx = _knn(x, k)
    x_t = jnp.transpose(x, (0, 2, 1))
    flat = x_t.reshape(B * N, C)
    idxf = (idx + (jnp.arange(B) * N)[:, None, None]).reshape(-1).astype(jnp.int32)
    Dp = max(16, -(-C // 16) * 16)
    flat_p = jnp.pad(flat, ((0, 0), (0, Dp - C)))
    feature = _sc_gather(flat_p, idxf)[:, :C].reshape(B, N, k, C)
    xr = jnp.broadcast_to(x_t[:, :, None, :], (B, N, k, C))
    d = feature - xr
    ded3 = jnp.sum(d * d, axis=-1, keepdims=True)
    x_glo = jnp.concatenate([jnp.sum(x, axis=1), jnp.mean(x, axis=1)], axis=-1)
    if firstlayer:
        f = jnp.concatenate([ded3, d, xr, feature], axis=3)
    else:
        f = jnp.concatenate([d, xr], axis=3)
    f = jnp.transpose(f, (0, 3, 1, 2))
    return x[:, :, :, None], f, x_glo


def _lab(xq, xk, p):
    q = _lrelu(_bn(jnp.einsum('oc,bcnk->bonk', p['Wk'], xq), p['gk'], p['bk'], 1))
    v = _lrelu(_bn(jnp.einsum('oc,bcnk->bonk', p['Wv'], xk), p['gv'], p['bv'], 1))
    s = _lrelu(_bn(jnp.einsum('oc,bcnk->bonk', p['Ws'], q + v), p['gs'], p['bs'], 1))
    s = jax.nn.softmax(s[:, 0], axis=2)
    return jnp.sum(v * s[:, None, :, :], axis=-1)


def _gab(xq, xk, p):
    values = _lrelu(_bn(xk @ p['W1'].T, p['g1'], p['b1'], 2))
    feats = values + xq[:, None, :]
    s = _lrelu(_bn(feats @ p['W2'].T, p['g2'], p['b2'], 2))
    s = _lrelu(_bn(s @ p['W3'].T, p['g3'], p['b3'], 2))
    s = _lrelu(_bn(s @ p['W4'].T, p['g4'], p['b4'], 2))
    s = jax.nn.softmax(s[..., 0], axis=1)
    return jnp.sum(values * s[:, :, None], axis=1)


def kernel(x, params, k):
    ver1, fea1, x_glo = _graph_feature(x, 20, True)
    x1 = _lab(ver1, fea1, params['lab1'])
    ver2, fea2, _ = _graph_feature(x1, 20, False)
    x2 = _lab(ver2, fea2, params['lab2'])
    ver3, fea3, _ = _graph_feature(x2, 20, False)
    x3 = _lab(ver3, fea3, params['lab3'])
    ver4, fea4, _ = _graph_feature(x3, 20, False)
    x4 = _lab(ver4, fea4, params['lab4'])
    xc = jnp.concatenate([x1, x2, x3, x4], axis=1)
    xk = jnp.transpose(xc, (0, 2, 1))
    g = _gab(x_glo, xk, params['gab'])
    h = _lrelu(_bn(g @ params['WL1'].T, params['g6'], params['b6'], 1))
    h = _lrelu(_bn(h @ params['WL2'].T + params['bL2'], params['g7'], params['b7'], 1))
    return h @ params['WL3'].T + params['bL3']
